# baseline (speedup 1.0000x reference)
import functools

import jax
import jax.numpy as jnp
from jax.experimental import pallas as pl
from jax.experimental.pallas import tpu as pltpu

_DEPTH = 1000
_BB = 16
_NBUF = 6


def _onehot_body(nblk, idx_ref, out_hbm, buf, sems):
    i = pl.program_id(0)
    slot = jax.lax.rem(i, _NBUF)

    @pl.when(i >= _NBUF)
    def _wait_prev():
        pltpu.make_async_copy(
            buf.at[slot],
            out_hbm.at[pl.ds((i - _NBUF) * _BB, _BB)],
            sems.at[slot],
        ).wait()

    idx = idx_ref[...]
    iota = jax.lax.broadcasted_iota(
        jnp.int32, (_BB, idx.shape[1], _DEPTH), 2)
    buf[slot] = jnp.where(idx[..., None] == iota, jnp.float32(1.0),
                          jnp.float32(0.0))

    pltpu.make_async_copy(
        buf.at[slot],
        out_hbm.at[pl.ds(i * _BB, _BB)],
        sems.at[slot],
    ).start()

    @pl.when(i == nblk - 1)
    def _drain():
        for s in range(nblk - _NBUF, nblk):
            pltpu.make_async_copy(
                buf.at[s % _NBUF],
                out_hbm.at[pl.ds(s * _BB, _BB)],
                sems.at[s % _NBUF],
            ).wait()


def kernel(indices):
    B, P = indices.shape
    indices = indices.astype(jnp.int32)
    nblk = B // _BB
    return pl.pallas_call(
        functools.partial(_onehot_body, nblk),
        grid=(nblk,),
        in_specs=[pl.BlockSpec((_BB, P), lambda i: (i, 0))],
        out_specs=pl.BlockSpec(memory_space=pltpu.MemorySpace.HBM),
        out_shape=jax.ShapeDtypeStruct((B, P, _DEPTH), jnp.float32),
        scratch_shapes=[
            pltpu.VMEM((_NBUF, _BB, P, _DEPTH), jnp.float32),
            pltpu.SemaphoreType.DMA((_NBUF,)),
        ],
    )(indices)
